# Initial kernel scaffold; baseline (speedup 1.0000x reference)
import jax
import jax.numpy as jnp
from jax.experimental import pallas as pl


def kernel(embedding, edge_index):
    raise NotImplementedError("write your pallas kernel here")


# speedup vs baseline: 1.5205x; 1.5205x over previous
import jax
import jax.numpy as jnp
from jax import lax
from jax.experimental import pallas as pl
from jax.experimental.pallas import tpu as pltpu
from jax.experimental.pallas import tpu_sc as plsc

_NUM_EDGES = 1_048_576
_EMBED_DIM = 32

_NC = 2
_NS = 16
_NW = _NC * _NS
_LANES = 16

_CHUNK = 512
_SUB = _CHUNK // 128
_EPW = _NUM_EDGES // _NW
_NCHUNK = _EPW // _CHUNK


def _tec_body(emb_hbm, sidx_hbm, didx_hbm, out_hbm,
              sidx_v, didx_v, srow_v, drow_v, out_v, sem):
    c = lax.axis_index("c")
    s = lax.axis_index("s")
    wid = s * _NC + c

    def chunk(ci, carry):
        pltpu.sync_copy(sidx_hbm.at[wid, ci], sidx_v)
        pltpu.sync_copy(didx_hbm.at[wid, ci], didx_v)
        cps = []
        for j in range(_SUB):
            cps.append(pltpu.async_copy(
                emb_hbm.at[sidx_v.at[j]],
                srow_v.at[pl.ds(j * 128, 128)], sem))
            cps.append(pltpu.async_copy(
                emb_hbm.at[didx_v.at[j]],
                drow_v.at[pl.ds(j * 128, 128)], sem))
        for cp in cps:
            cp.wait()

        def group(g, carry2):
            rows = g * _LANES + lax.iota(jnp.int32, _LANES)
            acc = jnp.zeros((_LANES,), jnp.float32)
            for d in range(_EMBED_DIM):
                col = jnp.full((_LANES,), d, jnp.int32)
                sv = plsc.load_gather(srow_v, [rows, col])
                dv = plsc.load_gather(drow_v, [rows, col])
                acc = acc + sv * dv
            out_v[pl.ds(g * _LANES, _LANES)] = acc
            return carry2

        lax.fori_loop(0, _CHUNK // _LANES, group, 0)
        pltpu.sync_copy(out_v, out_hbm.at[wid, ci])
        return carry

    lax.fori_loop(0, _NCHUNK, chunk, 0)


def kernel(embedding, edge_index):
    src = edge_index[0].astype(jnp.int32).reshape(_NW, _NCHUNK, _SUB, 128)
    dst = edge_index[1].astype(jnp.int32).reshape(_NW, _NCHUNK, _SUB, 128)
    run = pl.kernel(
        _tec_body,
        out_type=jax.ShapeDtypeStruct((_NW, _NCHUNK, _CHUNK), jnp.float32),
        mesh=plsc.VectorSubcoreMesh(core_axis_name="c", subcore_axis_name="s",
                                    num_cores=_NC, num_subcores=_NS),
        scratch_types=[
            pltpu.VMEM((_SUB, 128), jnp.int32),
            pltpu.VMEM((_SUB, 128), jnp.int32),
            pltpu.VMEM((_CHUNK, _EMBED_DIM), jnp.float32),
            pltpu.VMEM((_CHUNK, _EMBED_DIM), jnp.float32),
            pltpu.VMEM((_CHUNK,), jnp.float32),
            pltpu.SemaphoreType.DMA,
        ],
        compiler_params=pltpu.CompilerParams(needs_layout_passes=False,
                                             use_tc_tiling_on_sc=False),
    )
    scores = run(embedding, src, dst)
    return scores.reshape(_NUM_EDGES, 1)


# speedup vs baseline: 3.3106x; 2.1772x over previous
import jax
import jax.numpy as jnp
from jax import lax
from jax.experimental import pallas as pl
from jax.experimental.pallas import tpu as pltpu
from jax.experimental.pallas import tpu_sc as plsc

_NUM_EDGES = 1_048_576
_EMBED_DIM = 32

_NC = 2
_NS = 16
_NW = _NC * _NS
_LANES = 16

_CHUNK = 512
_SUB = _CHUNK // 128
_EPW = _NUM_EDGES // _NW
_NCHUNK = _EPW // _CHUNK


def _tec_body(emb_hbm, sidx_hbm, didx_hbm, out_hbm,
              sidx_v, didx_v, srow_v, drow_v, out_v, sem):
    c = lax.axis_index("c")
    s = lax.axis_index("s")
    wid = s * _NC + c

    def chunk(ci, carry):
        pltpu.sync_copy(sidx_hbm.at[wid, ci], sidx_v)
        pltpu.sync_copy(didx_hbm.at[wid, ci], didx_v)
        cps = []
        for j in range(_SUB):
            cps.append(pltpu.async_copy(
                emb_hbm.at[sidx_v.at[j]],
                srow_v.at[pl.ds(j * 128, 128)], sem))
            cps.append(pltpu.async_copy(
                emb_hbm.at[didx_v.at[j]],
                drow_v.at[pl.ds(j * 128, 128)], sem))
        for cp in cps:
            cp.wait()

        lane = lax.iota(jnp.int32, _LANES)

        def group(g, carry2):
            rows = g * _LANES + lane
            acc = jnp.zeros((_LANES,), jnp.float32)
            for d in range(_EMBED_DIM):
                col = (lane + d) & (_EMBED_DIM - 1)
                sv = plsc.load_gather(srow_v, [rows, col])
                dv = plsc.load_gather(drow_v, [rows, col])
                acc = acc + sv * dv
            out_v[pl.ds(g * _LANES, _LANES)] = acc
            return carry2

        lax.fori_loop(0, _CHUNK // _LANES, group, 0)
        pltpu.sync_copy(out_v, out_hbm.at[wid, ci])
        return carry

    lax.fori_loop(0, _NCHUNK, chunk, 0)


def kernel(embedding, edge_index):
    src = edge_index[0].astype(jnp.int32).reshape(_NW, _NCHUNK, _SUB, 128)
    dst = edge_index[1].astype(jnp.int32).reshape(_NW, _NCHUNK, _SUB, 128)
    run = pl.kernel(
        _tec_body,
        out_type=jax.ShapeDtypeStruct((_NW, _NCHUNK, _CHUNK), jnp.float32),
        mesh=plsc.VectorSubcoreMesh(core_axis_name="c", subcore_axis_name="s",
                                    num_cores=_NC, num_subcores=_NS),
        scratch_types=[
            pltpu.VMEM((_SUB, 128), jnp.int32),
            pltpu.VMEM((_SUB, 128), jnp.int32),
            pltpu.VMEM((_CHUNK, _EMBED_DIM), jnp.float32),
            pltpu.VMEM((_CHUNK, _EMBED_DIM), jnp.float32),
            pltpu.VMEM((_CHUNK,), jnp.float32),
            pltpu.SemaphoreType.DMA,
        ],
        compiler_params=pltpu.CompilerParams(needs_layout_passes=False,
                                             use_tc_tiling_on_sc=False),
    )
    scores = run(embedding, src, dst)
    return scores.reshape(_NUM_EDGES, 1)


# speedup vs baseline: 4.1655x; 1.2582x over previous
import jax
import jax.numpy as jnp
from jax import lax
from jax.experimental import pallas as pl
from jax.experimental.pallas import tpu as pltpu
from jax.experimental.pallas import tpu_sc as plsc

_NUM_EDGES = 1_048_576
_EMBED_DIM = 32

_NC = 2
_NS = 16
_NW = _NC * _NS
_LANES = 16

_CHUNK = 512
_SUB = _CHUNK // 128
_EPW = _NUM_EDGES // _NW
_NCHUNK = _EPW // _CHUNK
_NQUAD = _NCHUNK // 4


def _tec_body(emb_hbm, sidx_hbm, didx_hbm, out_hbm,
              si0, si1, si2, si3, di0, di1, di2, di3,
              srow0, srow1, drow0, drow1, out0, out1,
              gsem0, gsem1, isem0, isem1, isem2, isem3, osem0, osem1):
    c = lax.axis_index("c")
    s = lax.axis_index("s")
    wid = s * _NC + c

    sib = (si0, si1, si2, si3)
    dib = (di0, di1, di2, di3)
    isem = (isem0, isem1, isem2, isem3)
    srow = (srow0, srow1)
    drow = (drow0, drow1)
    outb = (out0, out1)
    gsem = (gsem0, gsem1)
    osem = (osem0, osem1)
    lane = lax.iota(jnp.int32, _LANES)

    def ifire(ci, slot):
        pltpu.async_copy(sidx_hbm.at[wid, ci], sib[slot], isem[slot])
        pltpu.async_copy(didx_hbm.at[wid, ci], dib[slot], isem[slot])

    def iwait(ci, slot):
        pltpu.make_async_copy(sidx_hbm.at[wid, ci], sib[slot], isem[slot]).wait()
        pltpu.make_async_copy(didx_hbm.at[wid, ci], dib[slot], isem[slot]).wait()

    def gfire(slot, rs):
        for j in range(_SUB):
            pltpu.async_copy(emb_hbm.at[sib[slot].at[j]],
                             srow[rs].at[pl.ds(j * 128, 128)], gsem[rs])
            pltpu.async_copy(emb_hbm.at[dib[slot].at[j]],
                             drow[rs].at[pl.ds(j * 128, 128)], gsem[rs])

    def gdrain(slot, rs):
        for j in range(_SUB):
            pltpu.make_async_copy(emb_hbm.at[sib[slot].at[j]],
                                  srow[rs].at[pl.ds(j * 128, 128)],
                                  gsem[rs]).wait()
            pltpu.make_async_copy(emb_hbm.at[dib[slot].at[j]],
                                  drow[rs].at[pl.ds(j * 128, 128)],
                                  gsem[rs]).wait()

    def owait(ci, rs):
        pltpu.make_async_copy(outb[rs], out_hbm.at[wid, ci], osem[rs]).wait()

    def compute(ci, rs):
        sr, dr, ov = srow[rs], drow[rs], outb[rs]

        def group(g, carry2):
            rows = g * _LANES + lane
            acc = jnp.zeros((_LANES,), jnp.float32)
            for d in range(_EMBED_DIM):
                col = (lane + d) & (_EMBED_DIM - 1)
                sv = plsc.load_gather(sr, [rows, col])
                dv = plsc.load_gather(dr, [rows, col])
                acc = acc + sv * dv
            ov[pl.ds(g * _LANES, _LANES)] = acc
            return carry2

        lax.fori_loop(0, _CHUNK // _LANES, group, 0)
        pltpu.async_copy(ov, out_hbm.at[wid, ci], osem[rs])

    for k in range(4):
        ifire(k, k)
    for k in range(2):
        iwait(k, k)
        gfire(k, k)

    def body(p, carry):
        base = p * 4
        for k in range(4):
            ci = base + k
            rs = k & 1
            gdrain(k, rs)
            if k < 2:
                @pl.when(p >= 1)
                def _():
                    owait(ci - 2, rs)
            else:
                owait(ci - 2, rs)
            compute(ci, rs)
            nslot = (k + 2) & 3
            if k < 2:
                iwait(ci + 2, nslot)
                gfire(nslot, rs)
            else:
                @pl.when(p < _NQUAD - 1)
                def _():
                    iwait(ci + 2, nslot)
                    gfire(nslot, rs)

            @pl.when(p < _NQUAD - 1)
            def _():
                ifire(ci + 4, k)
        return carry

    lax.fori_loop(0, _NQUAD, body, 0)

    owait(_NCHUNK - 2, 0)
    owait(_NCHUNK - 1, 1)


def kernel(embedding, edge_index):
    eidx = edge_index.astype(jnp.int32).reshape(2, _NW, _NCHUNK, _SUB, 128)
    run = pl.kernel(
        _tec_body,
        out_type=jax.ShapeDtypeStruct((_NW, _NCHUNK, _CHUNK), jnp.float32),
        mesh=plsc.VectorSubcoreMesh(core_axis_name="c", subcore_axis_name="s",
                                    num_cores=_NC, num_subcores=_NS),
        scratch_types=[
            pltpu.VMEM((_SUB, 128), jnp.int32),
            pltpu.VMEM((_SUB, 128), jnp.int32),
            pltpu.VMEM((_SUB, 128), jnp.int32),
            pltpu.VMEM((_SUB, 128), jnp.int32),
            pltpu.VMEM((_SUB, 128), jnp.int32),
            pltpu.VMEM((_SUB, 128), jnp.int32),
            pltpu.VMEM((_SUB, 128), jnp.int32),
            pltpu.VMEM((_SUB, 128), jnp.int32),
            pltpu.VMEM((_CHUNK, _EMBED_DIM), jnp.float32),
            pltpu.VMEM((_CHUNK, _EMBED_DIM), jnp.float32),
            pltpu.VMEM((_CHUNK, _EMBED_DIM), jnp.float32),
            pltpu.VMEM((_CHUNK, _EMBED_DIM), jnp.float32),
            pltpu.VMEM((_CHUNK,), jnp.float32),
            pltpu.VMEM((_CHUNK,), jnp.float32),
            pltpu.SemaphoreType.DMA,
            pltpu.SemaphoreType.DMA,
            pltpu.SemaphoreType.DMA,
            pltpu.SemaphoreType.DMA,
            pltpu.SemaphoreType.DMA,
            pltpu.SemaphoreType.DMA,
            pltpu.SemaphoreType.DMA,
            pltpu.SemaphoreType.DMA,
        ],
        compiler_params=pltpu.CompilerParams(needs_layout_passes=False,
                                             use_tc_tiling_on_sc=False),
    )
    scores = run(embedding, eidx[0], eidx[1])
    return scores.reshape(_NUM_EDGES, 1)


# speedup vs baseline: 4.1930x; 1.0066x over previous
import jax
import jax.numpy as jnp
from jax import lax
from jax.experimental import pallas as pl
from jax.experimental.pallas import tpu as pltpu
from jax.experimental.pallas import tpu_sc as plsc

_NUM_EDGES = 1_048_576
_EMBED_DIM = 32

_NC = 2
_NS = 16
_NW = _NC * _NS
_LANES = 16

_CHUNK = 512
_SUB = _CHUNK // 128
_EPW = _NUM_EDGES // _NW
_NCHUNK = _EPW // _CHUNK
_NQUAD = _NCHUNK // 4


def _tec_body(emb_hbm, eidx_hbm, out_hbm,
              si0, si1, si2, si3, di0, di1, di2, di3,
              srow0, srow1, drow0, drow1, out0, out1,
              gsem0, gsem1, isem0, isem1, isem2, isem3, osem0, osem1):
    c = lax.axis_index("c")
    s = lax.axis_index("s")
    wid = s * _NC + c

    sib = (si0, si1, si2, si3)
    dib = (di0, di1, di2, di3)
    isem = (isem0, isem1, isem2, isem3)
    srow = (srow0, srow1)
    drow = (drow0, drow1)
    outb = (out0, out1)
    gsem = (gsem0, gsem1)
    osem = (osem0, osem1)
    lane = lax.iota(jnp.int32, _LANES)

    def ifire(ci, slot):
        pltpu.async_copy(eidx_hbm.at[0, wid, ci], sib[slot], isem[slot])
        pltpu.async_copy(eidx_hbm.at[1, wid, ci], dib[slot], isem[slot])

    def iwait(ci, slot):
        pltpu.make_async_copy(eidx_hbm.at[0, wid, ci], sib[slot], isem[slot]).wait()
        pltpu.make_async_copy(eidx_hbm.at[1, wid, ci], dib[slot], isem[slot]).wait()

    def gfire(slot, rs):
        for j in range(_SUB):
            pltpu.async_copy(emb_hbm.at[sib[slot].at[j]],
                             srow[rs].at[pl.ds(j * 128, 128)], gsem[rs])
            pltpu.async_copy(emb_hbm.at[dib[slot].at[j]],
                             drow[rs].at[pl.ds(j * 128, 128)], gsem[rs])

    def gdrain(slot, rs):
        for j in range(_SUB):
            pltpu.make_async_copy(emb_hbm.at[sib[slot].at[j]],
                                  srow[rs].at[pl.ds(j * 128, 128)],
                                  gsem[rs]).wait()
            pltpu.make_async_copy(emb_hbm.at[dib[slot].at[j]],
                                  drow[rs].at[pl.ds(j * 128, 128)],
                                  gsem[rs]).wait()

    def owait(ci, rs):
        pltpu.make_async_copy(outb[rs], out_hbm.at[wid, ci], osem[rs]).wait()

    def compute(ci, rs):
        sr, dr, ov = srow[rs], drow[rs], outb[rs]

        def group(g, carry2):
            rows = g * _LANES + lane
            acc = jnp.zeros((_LANES,), jnp.float32)
            for d in range(_EMBED_DIM):
                col = (lane + d) & (_EMBED_DIM - 1)
                sv = plsc.load_gather(sr, [rows, col])
                dv = plsc.load_gather(dr, [rows, col])
                acc = acc + sv * dv
            ov[pl.ds(g * _LANES, _LANES)] = acc
            return carry2

        lax.fori_loop(0, _CHUNK // _LANES, group, 0)
        pltpu.async_copy(ov, out_hbm.at[wid, ci], osem[rs])

    for k in range(4):
        ifire(k, k)
    for k in range(2):
        iwait(k, k)
        gfire(k, k)

    def body(p, carry):
        base = p * 4
        for k in range(4):
            ci = base + k
            rs = k & 1
            gdrain(k, rs)
            if k < 2:
                @pl.when(p >= 1)
                def _():
                    owait(ci - 2, rs)
            else:
                owait(ci - 2, rs)
            compute(ci, rs)
            nslot = (k + 2) & 3
            if k < 2:
                iwait(ci + 2, nslot)
                gfire(nslot, rs)
            else:
                @pl.when(p < _NQUAD - 1)
                def _():
                    iwait(ci + 2, nslot)
                    gfire(nslot, rs)

            @pl.when(p < _NQUAD - 1)
            def _():
                ifire(ci + 4, k)
        return carry

    lax.fori_loop(0, _NQUAD, body, 0)

    owait(_NCHUNK - 2, 0)
    owait(_NCHUNK - 1, 1)


def kernel(embedding, edge_index):
    eidx = edge_index.astype(jnp.int32).reshape(2, _NW, _NCHUNK, _SUB, 128)
    run = pl.kernel(
        _tec_body,
        out_type=jax.ShapeDtypeStruct((_NW, _NCHUNK, _CHUNK), jnp.float32),
        mesh=plsc.VectorSubcoreMesh(core_axis_name="c", subcore_axis_name="s",
                                    num_cores=_NC, num_subcores=_NS),
        scratch_types=[
            pltpu.VMEM((_SUB, 128), jnp.int32),
            pltpu.VMEM((_SUB, 128), jnp.int32),
            pltpu.VMEM((_SUB, 128), jnp.int32),
            pltpu.VMEM((_SUB, 128), jnp.int32),
            pltpu.VMEM((_SUB, 128), jnp.int32),
            pltpu.VMEM((_SUB, 128), jnp.int32),
            pltpu.VMEM((_SUB, 128), jnp.int32),
            pltpu.VMEM((_SUB, 128), jnp.int32),
            pltpu.VMEM((_CHUNK, _EMBED_DIM), jnp.float32),
            pltpu.VMEM((_CHUNK, _EMBED_DIM), jnp.float32),
            pltpu.VMEM((_CHUNK, _EMBED_DIM), jnp.float32),
            pltpu.VMEM((_CHUNK, _EMBED_DIM), jnp.float32),
            pltpu.VMEM((_CHUNK,), jnp.float32),
            pltpu.VMEM((_CHUNK,), jnp.float32),
            pltpu.SemaphoreType.DMA,
            pltpu.SemaphoreType.DMA,
            pltpu.SemaphoreType.DMA,
            pltpu.SemaphoreType.DMA,
            pltpu.SemaphoreType.DMA,
            pltpu.SemaphoreType.DMA,
            pltpu.SemaphoreType.DMA,
            pltpu.SemaphoreType.DMA,
        ],
        compiler_params=pltpu.CompilerParams(needs_layout_passes=False,
                                             use_tc_tiling_on_sc=False),
    )
    scores = run(embedding, eidx)
    return scores.reshape(_NUM_EDGES, 1)
